# baseline (speedup 1.0000x reference)
import functools

import jax
import jax.numpy as jnp
from jax import lax
from jax.experimental import pallas as pl
from jax.experimental.pallas import tpu as pltpu
from jax.experimental.pallas import tpu_sc as plsc

_NUM_CORES = 2
_NUM_SUBCORES = 16
_NUM_WORKERS = _NUM_CORES * _NUM_SUBCORES


@functools.partial(jax.jit, static_argnames=("n",))
def _sc_copy(x_flat, n):
    per_w = n // _NUM_WORKERS

    def body(x_hbm, out_hbm):
        wid = lax.axis_index("s") * _NUM_CORES + lax.axis_index("c")
        base = wid * per_w
        pltpu.sync_copy(x_hbm.at[pl.ds(base, per_w)],
                        out_hbm.at[pl.ds(base, per_w)])

    mesh = plsc.VectorSubcoreMesh(core_axis_name="c", subcore_axis_name="s")
    return pl.kernel(
        body,
        out_type=jax.ShapeDtypeStruct((n,), jnp.float32),
        mesh=mesh,
    )(x_flat)


def kernel(x, buffer):
    n = x.size
    out = _sc_copy(x.reshape(n), n)
    return out.reshape(x.shape)


# speedup vs baseline: 7.7008x; 7.7008x over previous
import functools

import jax
import jax.numpy as jnp
from jax import lax
from jax.experimental import pallas as pl
from jax.experimental.pallas import tpu as pltpu
from jax.experimental.pallas import tpu_sc as plsc

_NUM_CORES = 2
_NUM_SUBCORES = 16
_NUM_WORKERS = _NUM_CORES * _NUM_SUBCORES
_NCHUNK = 4


@functools.partial(jax.jit, static_argnames=("n",))
def _sc_copy(x_flat, n):
    per_w = n // _NUM_WORKERS
    chunk = per_w // _NCHUNK

    def body(x_hbm, out_hbm, *scratch):
        bufs = scratch[:_NCHUNK]
        in_sems = scratch[_NCHUNK:2 * _NCHUNK]
        out_sems = scratch[2 * _NCHUNK:]
        wid = lax.axis_index("s") * _NUM_CORES + lax.axis_index("c")
        base = wid * per_w
        loads = []
        for i in range(_NCHUNK):
            loads.append(pltpu.async_copy(
                x_hbm.at[pl.ds(base + i * chunk, chunk)], bufs[i],
                in_sems[i]))
        stores = []
        for i in range(_NCHUNK):
            loads[i].wait()
            stores.append(pltpu.async_copy(
                bufs[i], out_hbm.at[pl.ds(base + i * chunk, chunk)],
                out_sems[i]))
        for s in stores:
            s.wait()

    mesh = plsc.VectorSubcoreMesh(core_axis_name="c", subcore_axis_name="s")
    return pl.kernel(
        body,
        out_type=jax.ShapeDtypeStruct((n,), jnp.float32),
        mesh=mesh,
        scratch_types=(
            [pltpu.VMEM((chunk,), jnp.float32)] * _NCHUNK
            + [pltpu.SemaphoreType.DMA] * (2 * _NCHUNK)
        ),
    )(x_flat)


def kernel(x, buffer):
    n = x.size
    out = _sc_copy(x.reshape(n), n)
    return out.reshape(x.shape)


# speedup vs baseline: 7.7804x; 1.0103x over previous
import functools

import jax
import jax.numpy as jnp
from jax import lax
from jax.experimental import pallas as pl
from jax.experimental.pallas import tpu as pltpu
from jax.experimental.pallas import tpu_sc as plsc

_NUM_CORES = 2
_NUM_SUBCORES = 16
_NUM_WORKERS = _NUM_CORES * _NUM_SUBCORES
_NCHUNK = 1


@functools.partial(jax.jit, static_argnames=("n",))
def _sc_copy(x_flat, n):
    per_w = n // _NUM_WORKERS
    chunk = per_w // _NCHUNK

    def body(x_hbm, out_hbm, *scratch):
        bufs = scratch[:_NCHUNK]
        in_sems = scratch[_NCHUNK:2 * _NCHUNK]
        out_sems = scratch[2 * _NCHUNK:]
        wid = lax.axis_index("s") * _NUM_CORES + lax.axis_index("c")
        base = wid * per_w
        loads = []
        for i in range(_NCHUNK):
            loads.append(pltpu.async_copy(
                x_hbm.at[pl.ds(base + i * chunk, chunk)], bufs[i],
                in_sems[i]))
        stores = []
        for i in range(_NCHUNK):
            loads[i].wait()
            stores.append(pltpu.async_copy(
                bufs[i], out_hbm.at[pl.ds(base + i * chunk, chunk)],
                out_sems[i]))
        for s in stores:
            s.wait()

    mesh = plsc.VectorSubcoreMesh(core_axis_name="c", subcore_axis_name="s")
    return pl.kernel(
        body,
        out_type=jax.ShapeDtypeStruct((n,), jnp.float32),
        mesh=mesh,
        scratch_types=(
            [pltpu.VMEM((chunk,), jnp.float32)] * _NCHUNK
            + [pltpu.SemaphoreType.DMA] * (2 * _NCHUNK)
        ),
    )(x_flat)


def kernel(x, buffer):
    n = x.size
    out = _sc_copy(x.reshape(n), n)
    return out.reshape(x.shape)
